# baseline (speedup 1.0000x reference)
import jax
import jax.numpy as jnp
from jax.experimental import pallas as pl
from jax.experimental.pallas import tpu as pltpu

jax.config.update("jax_enable_x64", True)

_N_TOKENS = 1024


def _body(scal_ref, x_ref, out_ref):
    min_val = scal_ref[0, 0]
    delta = scal_ref[0, 1]
    y = (x_ref[...] - min_val) / delta
    y = jnp.minimum(jnp.maximum(y, 0.0), jnp.float32(_N_TOKENS - 1))
    tok = y.astype(jnp.int32)
    out_ref[...] = jnp.concatenate([tok, jnp.zeros_like(tok)], axis=1)


def kernel(input, min_val, delta):
    m, n = input.shape
    bm = 256
    grid = (m // bm,)
    with jax.enable_x64(False):
        scal = jnp.stack([min_val.astype(jnp.float32),
                          delta.astype(jnp.float32)]).reshape(1, 2)
        out = pl.pallas_call(
            _body,
            grid=grid,
            in_specs=[
                pl.BlockSpec(memory_space=pltpu.SMEM),
                pl.BlockSpec((bm, n), lambda i: (i, 0)),
            ],
            out_specs=pl.BlockSpec((bm, 2 * n), lambda i: (i, 0)),
            out_shape=jax.ShapeDtypeStruct((m, 2 * n), jnp.int32),
        )(scal, input)
    return jax.lax.bitcast_convert_type(out.reshape(m, n, 2), jnp.int64)


# speedup vs baseline: 1.7639x; 1.7639x over previous
import jax
import jax.numpy as jnp
from jax import lax
from jax.experimental import pallas as pl
from jax.experimental.pallas import tpu as pltpu

jax.config.update("jax_enable_x64", True)

_N_TOKENS = 1024


def _body(scal_ref, x_ref, out_ref):
    min_val = scal_ref[0, 0]
    delta = scal_ref[0, 1]
    y = (x_ref[...] - min_val) / delta
    y = jnp.minimum(jnp.maximum(y, 0.0), jnp.float32(_N_TOKENS - 1))
    out_ref[...] = y.astype(jnp.uint32)


def kernel(input, min_val, delta):
    m, n = input.shape
    bm = 256
    grid = (m // bm,)
    scal = jnp.stack([min_val.astype(jnp.float32),
                      delta.astype(jnp.float32)]).reshape(1, 2)
    out = pl.pallas_call(
        _body,
        grid=grid,
        in_specs=[
            pl.BlockSpec((1, 2), lambda i: (jnp.int32(0), jnp.int32(0)),
                         memory_space=pltpu.SMEM),
            pl.BlockSpec((bm, n), lambda i: (jnp.int32(i), jnp.int32(0))),
        ],
        out_specs=pl.BlockSpec((bm, n),
                               lambda i: (jnp.int32(i), jnp.int32(0))),
        out_shape=jax.ShapeDtypeStruct((m, n), jnp.uint32),
    )(scal, input)
    return lax.bitcast_convert_type(out.astype(jnp.uint64), jnp.int64)
